# baseline (speedup 1.0000x reference)
import functools

import jax
import jax.numpy as jnp
from jax import lax
from jax.experimental import pallas as pl
from jax.experimental.pallas import tpu as pltpu
from jax.experimental.pallas import tpu_sc as plsc

N = 10000
E = 320000
D = 128
R = 200

NC = 2
NS = 16
EPT = E // NS
C = 48
EPTP = 20160
NCHUNK = EPTP // C
BLK = 20
NP = 10112
RPT = NP // NS


def _pack_edges(col, row, typ, nrm):
  pad = ((0, 0), (0, EPTP - EPT))
  col = jnp.pad(col.reshape(NS, EPT), pad).reshape(NS, NCHUNK, C)
  row = jnp.pad(row.reshape(NS, EPT), pad).reshape(NS, NCHUNK, C)
  typ = jnp.pad(typ.reshape(NS, EPT), pad).reshape(NS, NCHUNK, C)
  nrm = jnp.pad(lax.bitcast_convert_type(nrm, jnp.int32).reshape(NS, EPT),
                pad).reshape(NS, NCHUNK, C)
  return jnp.stack([col, row, typ, nrm], axis=2).reshape(-1)


def _sc_aggregate(H_all, rel_all, pk_all, zeros_tile):
  mesh = plsc.VectorSubcoreMesh(core_axis_name="c", subcore_axis_name="s")
  f32 = jnp.float32
  CW = 4 * C
  BW = BLK * CW
  SEGW = NS * NCHUNK * CW

  @functools.partial(
      pl.kernel,
      out_type=jax.ShapeDtypeStruct((4 * NP, D), f32),
      mesh=mesh,
      compiler_params=pltpu.CompilerParams(needs_layout_passes=False),
      scratch_types=[
          pltpu.VMEM((BW,), jnp.int32),
          pltpu.VMEM((C,), jnp.int32),
          pltpu.VMEM((C,), jnp.int32),
          pltpu.VMEM((C,), jnp.int32),
          pltpu.VMEM((C,), jnp.int32),
          pltpu.VMEM((C,), jnp.int32),
          pltpu.VMEM((C,), jnp.int32),
          pltpu.VMEM((C,), f32),
          pltpu.VMEM((C,), f32),
          pltpu.VMEM((C, D), f32),
          pltpu.VMEM((C, D), f32),
          pltpu.VMEM((C, D), f32),
          pltpu.VMEM((R, D), f32),
          pltpu.VMEM_SHARED((NP, D), f32),
          pltpu.SemaphoreType.DMA,
          pltpu.SemaphoreType.DMA,
          pltpu.SemaphoreType.DMA,
      ],
  )
  def sc_kernel(h_hbm, rel_hbm, pk_hbm, z_hbm, out_hbm,
                iblk, row0, row1, col0, col1, typ0, typ1, nrm0, nrm1,
                h0, h1, msg, rel_l, acc, sh0, sh1, ss):
    cid = lax.axis_index("c")
    sid = lax.axis_index("s")
    iota16 = lax.broadcasted_iota(jnp.int32, (16,), 0)
    rows = (row0, row1)
    cols = (col0, col1)
    typs = (typ0, typ1)
    nrms = (nrm0, nrm1)
    hbufs = (h0, h1)
    hsems = (sh0, sh1)

    def do_pass(p, _):
      seg = cid * 2 + p
      pk_off = seg * SEGW
      rel_off = jnp.where(seg == 0, 0, jnp.where(seg == 2, 2 * R, R))
      out_off = seg * NP
      pltpu.sync_copy(rel_hbm.at[pl.ds(rel_off, R)], rel_l)
      pltpu.sync_copy(z_hbm, acc.at[pl.ds(sid * RPT, RPT)])
      plsc.subcore_barrier()

      def prep(j, b):
        @pl.when(lax.rem(j, BLK) == 0)
        def _():
          blk_off = pk_off + (sid * NCHUNK + j) * CW
          pltpu.sync_copy(pk_hbm.at[pl.ds(blk_off, BW)], iblk)
        off = lax.rem(j, BLK) * CW
        for jj in range(C // 16):
          c16 = plsc.load_gather(iblk, [iota16 + (off + jj * 16)])
          cols[b][pl.ds(jj * 16, 16)] = c16
          r16 = plsc.load_gather(iblk, [iota16 + (off + C + jj * 16)])
          rows[b][pl.ds(jj * 16, 16)] = r16
          t16 = plsc.load_gather(iblk, [iota16 + (off + 2 * C + jj * 16)])
          typs[b][pl.ds(jj * 16, 16)] = t16
          n16 = plsc.load_gather(iblk, [iota16 + (off + 3 * C + jj * 16)])
          nrms[b][pl.ds(jj * 16, 16)] = plsc.bitcast(n16, f32)
        pltpu.async_copy(h_hbm.at[cols[b]], hbufs[b], hsems[b])

      def compute(k, b):
        pltpu.make_async_copy(h_hbm.at[cols[b]], hbufs[b], hsems[b]).wait()
        for g in range(C // 16):
          e16 = iota16 + g * 16
          n16 = plsc.load_gather(nrms[b], [e16])
          t16 = plsc.load_gather(typs[b], [e16])

          @plsc.parallel_loop(0, D, step=16, unroll=1,
                              carry=jnp.zeros((16,), jnp.int32))
          def _(dc, d16):
            for _u in range(16):
              h16 = plsc.load_gather(hbufs[b], [e16, d16])
              r16 = plsc.load_gather(rel_l, [t16, d16])
              v = (h16 + r16) * n16
              plsc.store_scatter(msg, [e16, d16], v)
              d16 = d16 + 1
            return d16

        pltpu.async_copy(msg, acc.at[rows[b]], ss, add=True)

      prep(jnp.int32(0), 0)

      def pair(k2, carry):
        for b in range(2):
          k = k2 * 2 + b
          nb = 1 - b

          @pl.when(k >= 1)
          def _():
            pltpu.make_async_copy(msg, acc.at[rows[b]], ss).wait()

          @pl.when(k + 1 < NCHUNK)
          def _():
            prep(k + 1, nb)
          compute(k, b)
        return carry

      lax.fori_loop(0, NCHUNK // 2, pair, 0)
      pltpu.make_async_copy(msg, acc.at[rows[1]], ss).wait()
      plsc.subcore_barrier()
      pltpu.sync_copy(acc.at[pl.ds(sid * RPT, RPT)],
                      out_hbm.at[pl.ds(out_off + sid * RPT, RPT)])
      plsc.subcore_barrier()
      return _

    lax.fori_loop(0, 2, do_pass, 0)

  return sc_kernel(H_all, rel_all, pk_all, zeros_tile)


def _dot_t(x, w):
  return lax.dot_general(x, w, (((1,), (1,)), ((), ())),
                         preferred_element_type=jnp.float32)


def _tc_combine_body(aic, air, aoc, aor, hc, hr, win, wout, wloop, lrc, lrr,
                     hnc_o, hnr_o):
  w_in = win[...]
  w_out = wout[...]
  w_loop = wloop[...]
  x = lrr[...]
  sp = jnp.maximum(x, 0.0) + jnp.log(1.0 + jnp.exp(-jnp.abs(x)))
  c3 = (_dot_t(aic[...], w_in) + _dot_t(aoc[...], w_out)
        + _dot_t(hc[...] + lrc[...], w_loop))
  r3 = (_dot_t(air[...], jnp.abs(w_in)) + _dot_t(aor[...], jnp.abs(w_out))
        + _dot_t(hr[...] + sp, jnp.abs(w_loop)))
  c = c3 * (1.0 / 3.0)
  r = r3 * (1.0 / 3.0)
  lo = jnp.maximum(c - r, 0.0)
  hi = jnp.maximum(c + r, 0.0)
  hnc_o[...] = (hi + lo) * 0.5
  hnr_o[...] = (hi - lo) * 0.5


def _tc_combine(a_in_c, a_in_r, a_out_c, a_out_r, H_c, H_r,
                W_in, W_out, W_loop, loop_rel_c, loop_rel_r):
  blk = 2000
  grid = (N // blk,)
  row_spec = pl.BlockSpec((blk, D), lambda i: (i, 0))
  w_spec = pl.BlockSpec((D, D), lambda i: (0, 0))
  v_spec = pl.BlockSpec((1, D), lambda i: (0, 0))
  return pl.pallas_call(
      _tc_combine_body,
      grid=grid,
      in_specs=[row_spec] * 6 + [w_spec] * 3 + [v_spec] * 2,
      out_specs=[row_spec, row_spec],
      out_shape=[jax.ShapeDtypeStruct((N, D), jnp.float32)] * 2,
  )(a_in_c, a_in_r, a_out_c, a_out_r, H_c, H_r, W_in, W_out, W_loop,
    loop_rel_c, loop_rel_r)


def _tc_rel_body(rc, rr, wr, orc_o, orr_o):
  w = wr[...]
  orc_o[...] = _dot_t(rc[...], w)
  orr_o[...] = _dot_t(rr[...], jnp.abs(w))


def _tc_rel(rel_c, rel_r, W_rel):
  return pl.pallas_call(
      _tc_rel_body,
      out_shape=[jax.ShapeDtypeStruct((R, D), jnp.float32)] * 2,
  )(rel_c, rel_r, W_rel)


def kernel(H_c, H_r, rel_c, rel_r, in_row, in_col, in_type, in_norm,
           out_row, out_col, out_type, out_norm, loop_row, loop_col,
           W_in, W_out, W_loop, W_rel, loop_rel_c, loop_rel_r):
  zeros_tile = jnp.zeros((RPT, D), jnp.float32)
  in_row = in_row.astype(jnp.int32)
  in_col = in_col.astype(jnp.int32)
  in_type = in_type.astype(jnp.int32)
  out_row = out_row.astype(jnp.int32)
  out_col = out_col.astype(jnp.int32)
  out_type = out_type.astype(jnp.int32)
  H_all = jnp.concatenate([H_c, H_r], axis=0)
  rel_all = jnp.concatenate([rel_c, rel_r, -rel_c], axis=0)
  pk_all = jnp.concatenate([
      _pack_edges(in_col, in_row, in_type, in_norm),
      _pack_edges(in_col + N, in_row, in_type, in_norm),
      _pack_edges(out_col, out_row, out_type, out_norm),
      _pack_edges(out_col + N, out_row, out_type, out_norm)])
  outs = _sc_aggregate(H_all, rel_all, pk_all, zeros_tile)
  a_in_c = outs[:N]
  a_in_r = outs[NP:NP + N]
  a_out_c = outs[2 * NP:2 * NP + N]
  a_out_r = outs[3 * NP:3 * NP + N]
  Hn_c, Hn_r = _tc_combine(a_in_c, a_in_r, a_out_c, a_out_r, H_c, H_r,
                           W_in, W_out, W_loop, loop_rel_c, loop_rel_r)
  new_rel_c, new_rel_r = _tc_rel(rel_c, rel_r, W_rel)
  return Hn_c, Hn_r, new_rel_c, new_rel_r


# speedup vs baseline: 4.6030x; 4.6030x over previous
import functools

import jax
import jax.numpy as jnp
from jax import lax
from jax.experimental import pallas as pl
from jax.experimental.pallas import tpu as pltpu
from jax.experimental.pallas import tpu_sc as plsc

N = 10000
E = 320000
D = 128
R = 200

NC = 2
NS = 16
EPT = E // NS
C = 48
EPTP = 20160
NCHUNK = EPTP // C
BLK = 20
NP = 10112
RPT = NP // NS


def _pack_edges(col, row, typ, nrm):
  pad = ((0, 0), (0, EPTP - EPT))
  col = jnp.pad(col.reshape(NS, EPT), pad).reshape(NS, NCHUNK, C)
  row = jnp.pad(row.reshape(NS, EPT), pad).reshape(NS, NCHUNK, C)
  typ = jnp.pad(typ.reshape(NS, EPT), pad).reshape(NS, NCHUNK, C)
  nrm = jnp.pad(lax.bitcast_convert_type(nrm, jnp.int32).reshape(NS, EPT),
                pad).reshape(NS, NCHUNK, C)
  return jnp.stack([col, row, typ, nrm], axis=2).reshape(-1)


def _sc_aggregate(H_all, rel_all, pk_all, zeros_tile):
  mesh = plsc.VectorSubcoreMesh(core_axis_name="c", subcore_axis_name="s")
  f32 = jnp.float32
  CW = 4 * C
  BW = BLK * CW
  SEGW = NS * NCHUNK * CW

  @functools.partial(
      pl.kernel,
      out_type=jax.ShapeDtypeStruct((4 * NP, D), f32),
      mesh=mesh,
      compiler_params=pltpu.CompilerParams(needs_layout_passes=False),
      scratch_types=[
          pltpu.VMEM((BW,), jnp.int32),
          pltpu.VMEM((C,), jnp.int32),
          pltpu.VMEM((C,), jnp.int32),
          pltpu.VMEM((C,), jnp.int32),
          pltpu.VMEM((C,), jnp.int32),
          pltpu.VMEM((C + 16,), jnp.int32),
          pltpu.VMEM((C + 16,), jnp.int32),
          pltpu.VMEM((C + 16,), f32),
          pltpu.VMEM((C + 16,), f32),
          pltpu.VMEM((C, D), f32),
          pltpu.VMEM((C, D), f32),
          pltpu.VMEM((C, D), f32),
          pltpu.VMEM((R, D), f32),
          pltpu.VMEM_SHARED((NP, D), f32),
          pltpu.SemaphoreType.DMA,
          pltpu.SemaphoreType.DMA,
          pltpu.SemaphoreType.DMA,
      ],
  )
  def sc_kernel(h_hbm, rel_hbm, pk_hbm, z_hbm, out_hbm,
                iblk, row0, row1, col0, col1, typ0, typ1, nrm0, nrm1,
                h0, h1, msg, rel_l, acc, sh0, sh1, ss):
    cid = lax.axis_index("c")
    sid = lax.axis_index("s")
    iota16 = lax.broadcasted_iota(jnp.int32, (16,), 0)
    rows = (row0, row1)
    cols = (col0, col1)
    typs = (typ0, typ1)
    nrms = (nrm0, nrm1)
    hbufs = (h0, h1)
    hsems = (sh0, sh1)

    def do_pass(p, _):
      seg = cid * 2 + p
      pk_off = seg * SEGW
      rel_off = jnp.where(seg == 0, 0, jnp.where(seg == 2, 2 * R, R))
      out_off = seg * NP
      pltpu.sync_copy(rel_hbm.at[pl.ds(rel_off, R)], rel_l)
      pltpu.sync_copy(z_hbm, acc.at[pl.ds(sid * RPT, RPT)])
      plsc.subcore_barrier()

      def prep(j, b):
        @pl.when(lax.rem(j, BLK) == 0)
        def _():
          blk_off = pk_off + (sid * NCHUNK + j) * CW
          pltpu.sync_copy(pk_hbm.at[pl.ds(blk_off, BW)], iblk)
        off = lax.rem(j, BLK) * CW
        for jj in range(C // 16):
          c16 = plsc.load_gather(iblk, [iota16 + (off + jj * 16)])
          cols[b][pl.ds(jj * 16, 16)] = c16
          r16 = plsc.load_gather(iblk, [iota16 + (off + C + jj * 16)])
          rows[b][pl.ds(jj * 16, 16)] = r16
          t16 = plsc.load_gather(iblk, [iota16 + (off + 2 * C + jj * 16)])
          typs[b][pl.ds(jj * 16, 16)] = t16
          n16 = plsc.load_gather(iblk, [iota16 + (off + 3 * C + jj * 16)])
          nrms[b][pl.ds(jj * 16, 16)] = plsc.bitcast(n16, f32)
        pltpu.async_copy(h_hbm.at[cols[b]], hbufs[b], hsems[b])

      def compute(k, b):
        pltpu.make_async_copy(h_hbm.at[cols[b]], hbufs[b], hsems[b]).wait()

        @plsc.parallel_loop(0, C, step=1, unroll=2)
        def _(e):
          n16 = jnp.full((16,), nrms[b][pl.ds(e, 16)][0], f32)
          t = typs[b][pl.ds(e, 16)][0]
          for dc in range(D // 16):
            h16 = hbufs[b][e, pl.ds(dc * 16, 16)]
            r16 = rel_l[t, pl.ds(dc * 16, 16)]
            msg[e, pl.ds(dc * 16, 16)] = (h16 + r16) * n16

        pltpu.async_copy(msg, acc.at[rows[b]], ss, add=True)

      prep(jnp.int32(0), 0)

      def pair(k2, carry):
        for b in range(2):
          k = k2 * 2 + b
          nb = 1 - b

          @pl.when(k >= 1)
          def _():
            pltpu.make_async_copy(msg, acc.at[rows[b]], ss).wait()

          @pl.when(k + 1 < NCHUNK)
          def _():
            prep(k + 1, nb)
          compute(k, b)
        return carry

      lax.fori_loop(0, NCHUNK // 2, pair, 0)
      pltpu.make_async_copy(msg, acc.at[rows[1]], ss).wait()
      plsc.subcore_barrier()
      pltpu.sync_copy(acc.at[pl.ds(sid * RPT, RPT)],
                      out_hbm.at[pl.ds(out_off + sid * RPT, RPT)])
      plsc.subcore_barrier()
      return _

    lax.fori_loop(0, 2, do_pass, 0)

  return sc_kernel(H_all, rel_all, pk_all, zeros_tile)


def _dot_t(x, w):
  return lax.dot_general(x, w, (((1,), (1,)), ((), ())),
                         preferred_element_type=jnp.float32)


def _tc_combine_body(aic, air, aoc, aor, hc, hr, win, wout, wloop, lrc, lrr,
                     hnc_o, hnr_o):
  w_in = win[...]
  w_out = wout[...]
  w_loop = wloop[...]
  x = lrr[...]
  sp = jnp.maximum(x, 0.0) + jnp.log(1.0 + jnp.exp(-jnp.abs(x)))
  c3 = (_dot_t(aic[...], w_in) + _dot_t(aoc[...], w_out)
        + _dot_t(hc[...] + lrc[...], w_loop))
  r3 = (_dot_t(air[...], jnp.abs(w_in)) + _dot_t(aor[...], jnp.abs(w_out))
        + _dot_t(hr[...] + sp, jnp.abs(w_loop)))
  c = c3 * (1.0 / 3.0)
  r = r3 * (1.0 / 3.0)
  lo = jnp.maximum(c - r, 0.0)
  hi = jnp.maximum(c + r, 0.0)
  hnc_o[...] = (hi + lo) * 0.5
  hnr_o[...] = (hi - lo) * 0.5


def _tc_combine(a_in_c, a_in_r, a_out_c, a_out_r, H_c, H_r,
                W_in, W_out, W_loop, loop_rel_c, loop_rel_r):
  blk = 2000
  grid = (N // blk,)
  row_spec = pl.BlockSpec((blk, D), lambda i: (i, 0))
  w_spec = pl.BlockSpec((D, D), lambda i: (0, 0))
  v_spec = pl.BlockSpec((1, D), lambda i: (0, 0))
  return pl.pallas_call(
      _tc_combine_body,
      grid=grid,
      in_specs=[row_spec] * 6 + [w_spec] * 3 + [v_spec] * 2,
      out_specs=[row_spec, row_spec],
      out_shape=[jax.ShapeDtypeStruct((N, D), jnp.float32)] * 2,
  )(a_in_c, a_in_r, a_out_c, a_out_r, H_c, H_r, W_in, W_out, W_loop,
    loop_rel_c, loop_rel_r)


def _tc_rel_body(rc, rr, wr, orc_o, orr_o):
  w = wr[...]
  orc_o[...] = _dot_t(rc[...], w)
  orr_o[...] = _dot_t(rr[...], jnp.abs(w))


def _tc_rel(rel_c, rel_r, W_rel):
  return pl.pallas_call(
      _tc_rel_body,
      out_shape=[jax.ShapeDtypeStruct((R, D), jnp.float32)] * 2,
  )(rel_c, rel_r, W_rel)


def kernel(H_c, H_r, rel_c, rel_r, in_row, in_col, in_type, in_norm,
           out_row, out_col, out_type, out_norm, loop_row, loop_col,
           W_in, W_out, W_loop, W_rel, loop_rel_c, loop_rel_r):
  zeros_tile = jnp.zeros((RPT, D), jnp.float32)
  in_row = in_row.astype(jnp.int32)
  in_col = in_col.astype(jnp.int32)
  in_type = in_type.astype(jnp.int32)
  out_row = out_row.astype(jnp.int32)
  out_col = out_col.astype(jnp.int32)
  out_type = out_type.astype(jnp.int32)
  H_all = jnp.concatenate([H_c, H_r], axis=0)
  rel_all = jnp.concatenate([rel_c, rel_r, -rel_c], axis=0)
  pk_all = jnp.concatenate([
      _pack_edges(in_col, in_row, in_type, in_norm),
      _pack_edges(in_col + N, in_row, in_type, in_norm),
      _pack_edges(out_col, out_row, out_type, out_norm),
      _pack_edges(out_col + N, out_row, out_type, out_norm)])
  outs = _sc_aggregate(H_all, rel_all, pk_all, zeros_tile)
  a_in_c = outs[:N]
  a_in_r = outs[NP:NP + N]
  a_out_c = outs[2 * NP:2 * NP + N]
  a_out_r = outs[3 * NP:3 * NP + N]
  Hn_c, Hn_r = _tc_combine(a_in_c, a_in_r, a_out_c, a_out_r, H_c, H_r,
                           W_in, W_out, W_loop, loop_rel_c, loop_rel_r)
  new_rel_c, new_rel_r = _tc_rel(rel_c, rel_r, W_rel)
  return Hn_c, Hn_r, new_rel_c, new_rel_r


# speedup vs baseline: 5.1765x; 1.1246x over previous
import functools

import jax
import jax.numpy as jnp
from jax import lax
from jax.experimental import pallas as pl
from jax.experimental.pallas import tpu as pltpu
from jax.experimental.pallas import tpu_sc as plsc

N = 10000
E = 320000
D = 128
R = 200

NC = 2
NS = 16
EPT = E // NS
C = 48
EPTP = 20160
NCHUNK = EPTP // C
BLK = 20
NP = 10112
RPT = NP // NS


def _pad_edges(x):
  return jnp.pad(x.reshape(NS, EPT), ((0, 0), (0, EPTP - EPT))).reshape(-1)


def _sc_aggregate(H_all, rel_all, icol, irow, ityp, inrm,
                  ocol, orow, otyp, onrm, zeros_tile):
  mesh = plsc.VectorSubcoreMesh(core_axis_name="c", subcore_axis_name="s")
  f32 = jnp.float32
  BW = BLK * C

  @functools.partial(
      pl.kernel,
      out_type=jax.ShapeDtypeStruct((4 * NP, D), f32),
      mesh=mesh,
      compiler_params=pltpu.CompilerParams(needs_layout_passes=False),
      scratch_types=[
          pltpu.VMEM((BW,), jnp.int32),
          pltpu.VMEM((BW,), jnp.int32),
          pltpu.VMEM((BW,), jnp.int32),
          pltpu.VMEM((BW,), f32),
          pltpu.VMEM((C,), jnp.int32),
          pltpu.VMEM((C,), jnp.int32),
          pltpu.VMEM((C,), jnp.int32),
          pltpu.VMEM((C,), jnp.int32),
          pltpu.VMEM((C + 16,), jnp.int32),
          pltpu.VMEM((C + 16,), jnp.int32),
          pltpu.VMEM((C + 16,), f32),
          pltpu.VMEM((C + 16,), f32),
          pltpu.VMEM((C, D), f32),
          pltpu.VMEM((C, D), f32),
          pltpu.VMEM((C, D), f32),
          pltpu.VMEM((R, D), f32),
          pltpu.VMEM_SHARED((NP, D), f32),
          pltpu.SemaphoreType.DMA,
          pltpu.SemaphoreType.DMA,
          pltpu.SemaphoreType.DMA,
      ],
  )
  def sc_kernel(h_hbm, rel_hbm, icol_hbm, irow_hbm, ityp_hbm, inrm_hbm,
                ocol_hbm, orow_hbm, otyp_hbm, onrm_hbm, z_hbm, out_hbm,
                cblk, rblk, tblk, nblk, row0, row1, col0, col1,
                typ0, typ1, nrm0, nrm1, h0, h1, msg, rel_l, acc,
                sh0, sh1, ss):
    cid = lax.axis_index("c")
    sid = lax.axis_index("s")
    iota16 = lax.broadcasted_iota(jnp.int32, (16,), 0)
    rows = (row0, row1)
    cols = (col0, col1)
    typs = (typ0, typ1)
    nrms = (nrm0, nrm1)
    hbufs = (h0, h1)
    hsems = (sh0, sh1)

    def body(p, col_hbm, row_hbm, typ_hbm, nrm_hbm):
      seg = cid * 2 + p
      rel_off = jnp.where(seg == 0, 0, jnp.where(seg == 2, 2 * R, R))
      coloff = p * N
      out_off = seg * NP
      pltpu.sync_copy(rel_hbm.at[pl.ds(rel_off, R)], rel_l)
      pltpu.sync_copy(z_hbm, acc.at[pl.ds(sid * RPT, RPT)])
      plsc.subcore_barrier()

      def prep(j, b):
        @pl.when(lax.rem(j, BLK) == 0)
        def _():
          blk_off = (sid * NCHUNK + j) * C
          pltpu.sync_copy(col_hbm.at[pl.ds(blk_off, BW)], cblk)
          pltpu.sync_copy(row_hbm.at[pl.ds(blk_off, BW)], rblk)
          pltpu.sync_copy(typ_hbm.at[pl.ds(blk_off, BW)], tblk)
          pltpu.sync_copy(nrm_hbm.at[pl.ds(blk_off, BW)], nblk)
        off = lax.rem(j, BLK) * C
        for jj in range(C // 16):
          sl = pl.ds(off + jj * 16, 16)
          cols[b][pl.ds(jj * 16, 16)] = cblk[sl] + coloff
          rows[b][pl.ds(jj * 16, 16)] = rblk[sl]
          typs[b][pl.ds(jj * 16, 16)] = tblk[sl]
          nrms[b][pl.ds(jj * 16, 16)] = nblk[sl]
        pltpu.async_copy(h_hbm.at[cols[b]], hbufs[b], hsems[b])

      def compute(k, b):
        pltpu.make_async_copy(h_hbm.at[cols[b]], hbufs[b], hsems[b]).wait()

        @plsc.parallel_loop(0, C, step=1, unroll=2)
        def _(e):
          n16 = jnp.full((16,), nrms[b][pl.ds(e, 16)][0], f32)
          t = typs[b][pl.ds(e, 16)][0]
          for dc in range(D // 16):
            h16 = hbufs[b][e, pl.ds(dc * 16, 16)]
            r16 = rel_l[t, pl.ds(dc * 16, 16)]
            msg[e, pl.ds(dc * 16, 16)] = (h16 + r16) * n16

        pltpu.async_copy(msg, acc.at[rows[b]], ss, add=True)

      prep(jnp.int32(0), 0)

      def pair(k2, carry):
        for b in range(2):
          k = k2 * 2 + b
          nb = 1 - b

          @pl.when(k >= 1)
          def _():
            pltpu.make_async_copy(msg, acc.at[rows[b]], ss).wait()

          @pl.when(k + 1 < NCHUNK)
          def _():
            prep(k + 1, nb)
          compute(k, b)
        return carry

      lax.fori_loop(0, NCHUNK // 2, pair, 0)
      pltpu.make_async_copy(msg, acc.at[rows[1]], ss).wait()
      plsc.subcore_barrier()
      pltpu.sync_copy(acc.at[pl.ds(sid * RPT, RPT)],
                      out_hbm.at[pl.ds(out_off + sid * RPT, RPT)])
      plsc.subcore_barrier()

    def do_pass(p, carry):
      @pl.when(cid == 0)
      def _():
        body(p, icol_hbm, irow_hbm, ityp_hbm, inrm_hbm)

      @pl.when(cid == 1)
      def _():
        body(p, ocol_hbm, orow_hbm, otyp_hbm, onrm_hbm)
      return carry

    lax.fori_loop(0, 2, do_pass, 0)

  return sc_kernel(H_all, rel_all, icol, irow, ityp, inrm,
                   ocol, orow, otyp, onrm, zeros_tile)


def _dot_t(x, w):
  return lax.dot_general(x, w, (((1,), (1,)), ((), ())),
                         preferred_element_type=jnp.float32)


def _tc_combine_body(aic, air, aoc, aor, hc, hr, win, wout, wloop, lrc, lrr,
                     hnc_o, hnr_o):
  w_in = win[...]
  w_out = wout[...]
  w_loop = wloop[...]
  x = lrr[...]
  sp = jnp.maximum(x, 0.0) + jnp.log(1.0 + jnp.exp(-jnp.abs(x)))
  c3 = (_dot_t(aic[...], w_in) + _dot_t(aoc[...], w_out)
        + _dot_t(hc[...] + lrc[...], w_loop))
  r3 = (_dot_t(air[...], jnp.abs(w_in)) + _dot_t(aor[...], jnp.abs(w_out))
        + _dot_t(hr[...] + sp, jnp.abs(w_loop)))
  c = c3 * (1.0 / 3.0)
  r = r3 * (1.0 / 3.0)
  lo = jnp.maximum(c - r, 0.0)
  hi = jnp.maximum(c + r, 0.0)
  hnc_o[...] = (hi + lo) * 0.5
  hnr_o[...] = (hi - lo) * 0.5


def _tc_combine(a_in_c, a_in_r, a_out_c, a_out_r, H_c, H_r,
                W_in, W_out, W_loop, loop_rel_c, loop_rel_r):
  blk = 2000
  grid = (N // blk,)
  row_spec = pl.BlockSpec((blk, D), lambda i: (i, 0))
  w_spec = pl.BlockSpec((D, D), lambda i: (0, 0))
  v_spec = pl.BlockSpec((1, D), lambda i: (0, 0))
  return pl.pallas_call(
      _tc_combine_body,
      grid=grid,
      in_specs=[row_spec] * 6 + [w_spec] * 3 + [v_spec] * 2,
      out_specs=[row_spec, row_spec],
      out_shape=[jax.ShapeDtypeStruct((N, D), jnp.float32)] * 2,
  )(a_in_c, a_in_r, a_out_c, a_out_r, H_c, H_r, W_in, W_out, W_loop,
    loop_rel_c, loop_rel_r)


def _tc_rel_body(rc, rr, wr, orc_o, orr_o):
  w = wr[...]
  orc_o[...] = _dot_t(rc[...], w)
  orr_o[...] = _dot_t(rr[...], jnp.abs(w))


def _tc_rel(rel_c, rel_r, W_rel):
  return pl.pallas_call(
      _tc_rel_body,
      out_shape=[jax.ShapeDtypeStruct((R, D), jnp.float32)] * 2,
  )(rel_c, rel_r, W_rel)


def kernel(H_c, H_r, rel_c, rel_r, in_row, in_col, in_type, in_norm,
           out_row, out_col, out_type, out_norm, loop_row, loop_col,
           W_in, W_out, W_loop, W_rel, loop_rel_c, loop_rel_r):
  zeros_tile = jnp.zeros((RPT, D), jnp.float32)
  in_row = in_row.astype(jnp.int32)
  in_col = in_col.astype(jnp.int32)
  in_type = in_type.astype(jnp.int32)
  out_row = out_row.astype(jnp.int32)
  out_col = out_col.astype(jnp.int32)
  out_type = out_type.astype(jnp.int32)
  H_all = jnp.concatenate([H_c, H_r], axis=0)
  rel_all = jnp.concatenate([rel_c, rel_r, -rel_c], axis=0)
  outs = _sc_aggregate(
      H_all, rel_all,
      _pad_edges(in_col), _pad_edges(in_row), _pad_edges(in_type),
      _pad_edges(in_norm), _pad_edges(out_col), _pad_edges(out_row),
      _pad_edges(out_type), _pad_edges(out_norm), zeros_tile)
  a_in_c = outs[:N]
  a_in_r = outs[NP:NP + N]
  a_out_c = outs[2 * NP:2 * NP + N]
  a_out_r = outs[3 * NP:3 * NP + N]
  Hn_c, Hn_r = _tc_combine(a_in_c, a_in_r, a_out_c, a_out_r, H_c, H_r,
                           W_in, W_out, W_loop, loop_rel_c, loop_rel_r)
  new_rel_c, new_rel_r = _tc_rel(rel_c, rel_r, W_rel)
  return Hn_c, Hn_r, new_rel_c, new_rel_r


# speedup vs baseline: 5.2098x; 1.0064x over previous
import functools

import jax
import jax.numpy as jnp
from jax import lax
from jax.experimental import pallas as pl
from jax.experimental.pallas import tpu as pltpu
from jax.experimental.pallas import tpu_sc as plsc

N = 10000
E = 320000
D = 128
R = 200

NC = 2
NS = 16
EPT = E // NS
C = 48
EPTP = 20160
NCHUNK = EPTP // C
BLK = 20
NP = 10112
RPT = NP // NS


def _pad_edges(x):
  return jnp.pad(x.reshape(NS, EPT), ((0, 0), (0, EPTP - EPT))).reshape(-1)


def _sc_aggregate(H_all, rel_all, icol, irow, ityp, inrm,
                  ocol, orow, otyp, onrm, zeros_tile):
  mesh = plsc.VectorSubcoreMesh(core_axis_name="c", subcore_axis_name="s")
  f32 = jnp.float32
  BW = BLK * C

  @functools.partial(
      pl.kernel,
      out_type=jax.ShapeDtypeStruct((4 * NP, D), f32),
      mesh=mesh,
      compiler_params=pltpu.CompilerParams(needs_layout_passes=False),
      scratch_types=[
          pltpu.VMEM((BW,), jnp.int32),
          pltpu.VMEM((BW,), jnp.int32),
          pltpu.VMEM((BW,), jnp.int32),
          pltpu.VMEM((BW,), f32),
          pltpu.VMEM((C,), jnp.int32),
          pltpu.VMEM((C,), jnp.int32),
          pltpu.VMEM((C,), jnp.int32),
          pltpu.VMEM((C,), jnp.int32),
          pltpu.VMEM((C + 16,), jnp.int32),
          pltpu.VMEM((C + 16,), jnp.int32),
          pltpu.VMEM((C + 16,), f32),
          pltpu.VMEM((C + 16,), f32),
          pltpu.VMEM((C, D), f32),
          pltpu.VMEM((C, D), f32),
          pltpu.VMEM((C, D), f32),
          pltpu.VMEM((R, D), f32),
          pltpu.VMEM_SHARED((NP, D), f32),
          pltpu.SemaphoreType.DMA,
          pltpu.SemaphoreType.DMA,
          pltpu.SemaphoreType.DMA,
      ],
  )
  def sc_kernel(h_hbm, rel_hbm, icol_hbm, irow_hbm, ityp_hbm, inrm_hbm,
                ocol_hbm, orow_hbm, otyp_hbm, onrm_hbm, z_hbm, out_hbm,
                cblk, rblk, tblk, nblk, row0, row1, col0, col1,
                typ0, typ1, nrm0, nrm1, h0, h1, msg, rel_l, acc,
                sh0, sh1, ss):
    cid = lax.axis_index("c")
    sid = lax.axis_index("s")
    iota16 = lax.broadcasted_iota(jnp.int32, (16,), 0)
    rows = (row0, row1)
    cols = (col0, col1)
    typs = (typ0, typ1)
    nrms = (nrm0, nrm1)
    hbufs = (h0, h1)
    hsems = (sh0, sh1)

    def body(p, col_hbm, row_hbm, typ_hbm, nrm_hbm):
      seg = cid * 2 + p
      rel_off = jnp.where(seg == 0, 0, jnp.where(seg == 2, 2 * R, R))
      coloff = p * N
      out_off = seg * NP
      pltpu.sync_copy(rel_hbm.at[pl.ds(rel_off, R)], rel_l)
      pltpu.sync_copy(z_hbm, acc.at[pl.ds(sid * RPT, RPT)])
      plsc.subcore_barrier()

      def prep(j, b):
        @pl.when(lax.rem(j, BLK) == 0)
        def _():
          blk_off = (sid * NCHUNK + j) * C
          pltpu.sync_copy(col_hbm.at[pl.ds(blk_off, BW)], cblk)
          pltpu.sync_copy(row_hbm.at[pl.ds(blk_off, BW)], rblk)
          pltpu.sync_copy(typ_hbm.at[pl.ds(blk_off, BW)], tblk)
          pltpu.sync_copy(nrm_hbm.at[pl.ds(blk_off, BW)], nblk)
        off = lax.rem(j, BLK) * C
        for jj in range(C // 16):
          sl = pl.ds(off + jj * 16, 16)
          cols[b][pl.ds(jj * 16, 16)] = cblk[sl] + coloff
          rows[b][pl.ds(jj * 16, 16)] = rblk[sl]
          typs[b][pl.ds(jj * 16, 16)] = tblk[sl]
          nrms[b][pl.ds(jj * 16, 16)] = nblk[sl]
        pltpu.async_copy(h_hbm.at[cols[b]], hbufs[b], hsems[b])

      def compute(k, b):
        pltpu.make_async_copy(h_hbm.at[cols[b]], hbufs[b], hsems[b]).wait()

        @plsc.parallel_loop(0, C, step=1, unroll=4)
        def _(e):
          n16 = jnp.full((16,), nrms[b][pl.ds(e, 16)][0], f32)
          t = typs[b][pl.ds(e, 16)][0]
          for dc in range(D // 16):
            h16 = hbufs[b][e, pl.ds(dc * 16, 16)]
            r16 = rel_l[t, pl.ds(dc * 16, 16)]
            msg[e, pl.ds(dc * 16, 16)] = (h16 + r16) * n16

        pltpu.async_copy(msg, acc.at[rows[b]], ss, add=True)

      prep(jnp.int32(0), 0)

      def pair(k2, carry):
        for b in range(2):
          k = k2 * 2 + b
          nb = 1 - b

          @pl.when(k >= 1)
          def _():
            pltpu.make_async_copy(msg, acc.at[rows[b]], ss).wait()

          @pl.when(k + 1 < NCHUNK)
          def _():
            prep(k + 1, nb)
          compute(k, b)
        return carry

      lax.fori_loop(0, NCHUNK // 2, pair, 0)
      pltpu.make_async_copy(msg, acc.at[rows[1]], ss).wait()
      plsc.subcore_barrier()
      pltpu.sync_copy(acc.at[pl.ds(sid * RPT, RPT)],
                      out_hbm.at[pl.ds(out_off + sid * RPT, RPT)])
      plsc.subcore_barrier()

    def do_pass(p, carry):
      @pl.when(cid == 0)
      def _():
        body(p, icol_hbm, irow_hbm, ityp_hbm, inrm_hbm)

      @pl.when(cid == 1)
      def _():
        body(p, ocol_hbm, orow_hbm, otyp_hbm, onrm_hbm)
      return carry

    lax.fori_loop(0, 2, do_pass, 0)

  return sc_kernel(H_all, rel_all, icol, irow, ityp, inrm,
                   ocol, orow, otyp, onrm, zeros_tile)


def _dot_t(x, w):
  return lax.dot_general(x, w, (((1,), (1,)), ((), ())),
                         preferred_element_type=jnp.float32)


def _tc_combine_body(aic, air, aoc, aor, hc, hr, win, wout, wloop, lrc, lrr,
                     hnc_o, hnr_o):
  w_in = win[...]
  w_out = wout[...]
  w_loop = wloop[...]
  x = lrr[...]
  sp = jnp.maximum(x, 0.0) + jnp.log(1.0 + jnp.exp(-jnp.abs(x)))
  c3 = (_dot_t(aic[...], w_in) + _dot_t(aoc[...], w_out)
        + _dot_t(hc[...] + lrc[...], w_loop))
  r3 = (_dot_t(air[...], jnp.abs(w_in)) + _dot_t(aor[...], jnp.abs(w_out))
        + _dot_t(hr[...] + sp, jnp.abs(w_loop)))
  c = c3 * (1.0 / 3.0)
  r = r3 * (1.0 / 3.0)
  lo = jnp.maximum(c - r, 0.0)
  hi = jnp.maximum(c + r, 0.0)
  hnc_o[...] = (hi + lo) * 0.5
  hnr_o[...] = (hi - lo) * 0.5


def _tc_combine(a_in_c, a_in_r, a_out_c, a_out_r, H_c, H_r,
                W_in, W_out, W_loop, loop_rel_c, loop_rel_r):
  blk = 2000
  grid = (N // blk,)
  row_spec = pl.BlockSpec((blk, D), lambda i: (i, 0))
  w_spec = pl.BlockSpec((D, D), lambda i: (0, 0))
  v_spec = pl.BlockSpec((1, D), lambda i: (0, 0))
  return pl.pallas_call(
      _tc_combine_body,
      grid=grid,
      in_specs=[row_spec] * 6 + [w_spec] * 3 + [v_spec] * 2,
      out_specs=[row_spec, row_spec],
      out_shape=[jax.ShapeDtypeStruct((N, D), jnp.float32)] * 2,
  )(a_in_c, a_in_r, a_out_c, a_out_r, H_c, H_r, W_in, W_out, W_loop,
    loop_rel_c, loop_rel_r)


def _tc_rel_body(rc, rr, wr, orc_o, orr_o):
  w = wr[...]
  orc_o[...] = _dot_t(rc[...], w)
  orr_o[...] = _dot_t(rr[...], jnp.abs(w))


def _tc_rel(rel_c, rel_r, W_rel):
  return pl.pallas_call(
      _tc_rel_body,
      out_shape=[jax.ShapeDtypeStruct((R, D), jnp.float32)] * 2,
  )(rel_c, rel_r, W_rel)


def kernel(H_c, H_r, rel_c, rel_r, in_row, in_col, in_type, in_norm,
           out_row, out_col, out_type, out_norm, loop_row, loop_col,
           W_in, W_out, W_loop, W_rel, loop_rel_c, loop_rel_r):
  zeros_tile = jnp.zeros((RPT, D), jnp.float32)
  in_row = in_row.astype(jnp.int32)
  in_col = in_col.astype(jnp.int32)
  in_type = in_type.astype(jnp.int32)
  out_row = out_row.astype(jnp.int32)
  out_col = out_col.astype(jnp.int32)
  out_type = out_type.astype(jnp.int32)
  H_all = jnp.concatenate([H_c, H_r], axis=0)
  rel_all = jnp.concatenate([rel_c, rel_r, -rel_c], axis=0)
  outs = _sc_aggregate(
      H_all, rel_all,
      _pad_edges(in_col), _pad_edges(in_row), _pad_edges(in_type),
      _pad_edges(in_norm), _pad_edges(out_col), _pad_edges(out_row),
      _pad_edges(out_type), _pad_edges(out_norm), zeros_tile)
  a_in_c = outs[:N]
  a_in_r = outs[NP:NP + N]
  a_out_c = outs[2 * NP:2 * NP + N]
  a_out_r = outs[3 * NP:3 * NP + N]
  Hn_c, Hn_r = _tc_combine(a_in_c, a_in_r, a_out_c, a_out_r, H_c, H_r,
                           W_in, W_out, W_loop, loop_rel_c, loop_rel_r)
  new_rel_c, new_rel_r = _tc_rel(rel_c, rel_r, W_rel)
  return Hn_c, Hn_r, new_rel_c, new_rel_r


# speedup vs baseline: 5.5539x; 1.0660x over previous
import functools

import jax
import jax.numpy as jnp
from jax import lax
from jax.experimental import pallas as pl
from jax.experimental.pallas import tpu as pltpu
from jax.experimental.pallas import tpu_sc as plsc

N = 10000
E = 320000
D = 128
R = 200

NC = 2
NS = 16
EPT = E // NS
C = 48
EPTP = 20160
NCHUNK = EPTP // C
BLK = 20
NP = 10112
RPT = NP // NS


def _pad_edges(x):
  return jnp.pad(x.reshape(NS, EPT), ((0, 0), (0, EPTP - EPT))).reshape(-1)


def _sc_aggregate(H_all, icol, irow, inrm, ocol, orow, onrm, zeros_tile):
  mesh = plsc.VectorSubcoreMesh(core_axis_name="c", subcore_axis_name="s")
  f32 = jnp.float32
  BW = BLK * C

  @functools.partial(
      pl.kernel,
      out_type=jax.ShapeDtypeStruct((4 * NP, D), f32),
      mesh=mesh,
      compiler_params=pltpu.CompilerParams(needs_layout_passes=False),
      scratch_types=[
          pltpu.VMEM((BW,), jnp.int32),
          pltpu.VMEM((BW,), jnp.int32),
          pltpu.VMEM((BW,), f32),
          pltpu.VMEM((C,), jnp.int32),
          pltpu.VMEM((C,), jnp.int32),
          pltpu.VMEM((C,), jnp.int32),
          pltpu.VMEM((C,), jnp.int32),
          pltpu.VMEM((C + 16,), f32),
          pltpu.VMEM((C + 16,), f32),
          pltpu.VMEM((C, D), f32),
          pltpu.VMEM((C, D), f32),
          pltpu.VMEM((C, D), f32),
          pltpu.VMEM_SHARED((NP, D), f32),
          pltpu.SemaphoreType.DMA,
          pltpu.SemaphoreType.DMA,
          pltpu.SemaphoreType.DMA,
      ],
  )
  def sc_kernel(h_hbm, icol_hbm, irow_hbm, inrm_hbm,
                ocol_hbm, orow_hbm, onrm_hbm, z_hbm, out_hbm,
                cblk, rblk, nblk, row0, row1, col0, col1,
                nrm0, nrm1, h0, h1, msg, acc, sh0, sh1, ss):
    cid = lax.axis_index("c")
    sid = lax.axis_index("s")
    iota16 = lax.broadcasted_iota(jnp.int32, (16,), 0)
    rows = (row0, row1)
    cols = (col0, col1)
    nrms = (nrm0, nrm1)
    hbufs = (h0, h1)
    hsems = (sh0, sh1)

    def body(p, col_hbm, row_hbm, nrm_hbm):
      seg = cid * 2 + p
      coloff = p * N
      out_off = seg * NP
      pltpu.sync_copy(z_hbm, acc.at[pl.ds(sid * RPT, RPT)])
      plsc.subcore_barrier()

      def prep(j, b):
        @pl.when(lax.rem(j, BLK) == 0)
        def _():
          blk_off = (sid * NCHUNK + j) * C
          pltpu.sync_copy(col_hbm.at[pl.ds(blk_off, BW)], cblk)
          pltpu.sync_copy(row_hbm.at[pl.ds(blk_off, BW)], rblk)
          pltpu.sync_copy(nrm_hbm.at[pl.ds(blk_off, BW)], nblk)
        off = lax.rem(j, BLK) * C
        for jj in range(C // 16):
          sl = pl.ds(off + jj * 16, 16)
          cols[b][pl.ds(jj * 16, 16)] = cblk[sl] + coloff
          rows[b][pl.ds(jj * 16, 16)] = rblk[sl]
          nrms[b][pl.ds(jj * 16, 16)] = nblk[sl]
        pltpu.async_copy(h_hbm.at[cols[b]], hbufs[b], hsems[b])

      def compute(k, b):
        pltpu.make_async_copy(h_hbm.at[cols[b]], hbufs[b], hsems[b]).wait()

        @plsc.parallel_loop(0, C, step=1, unroll=4)
        def _(e):
          n16 = jnp.full((16,), nrms[b][pl.ds(e, 16)][0], f32)
          for dc in range(D // 16):
            h16 = hbufs[b][e, pl.ds(dc * 16, 16)]
            msg[e, pl.ds(dc * 16, 16)] = h16 * n16

        pltpu.async_copy(msg, acc.at[rows[b]], ss, add=True)

      prep(jnp.int32(0), 0)

      def pair(k2, carry):
        for b in range(2):
          k = k2 * 2 + b
          nb = 1 - b

          @pl.when(k >= 1)
          def _():
            pltpu.make_async_copy(msg, acc.at[rows[b]], ss).wait()

          @pl.when(k + 1 < NCHUNK)
          def _():
            prep(k + 1, nb)
          compute(k, b)
        return carry

      lax.fori_loop(0, NCHUNK // 2, pair, 0)
      pltpu.make_async_copy(msg, acc.at[rows[1]], ss).wait()
      plsc.subcore_barrier()
      pltpu.sync_copy(acc.at[pl.ds(sid * RPT, RPT)],
                      out_hbm.at[pl.ds(out_off + sid * RPT, RPT)])
      plsc.subcore_barrier()

    def do_pass(p, carry):
      @pl.when(cid == 0)
      def _():
        body(p, icol_hbm, irow_hbm, inrm_hbm)

      @pl.when(cid == 1)
      def _():
        body(p, ocol_hbm, orow_hbm, onrm_hbm)
      return carry

    lax.fori_loop(0, 2, do_pass, 0)

  return sc_kernel(H_all, icol, irow, inrm,
                   ocol, orow, onrm, zeros_tile)


NPS = 10240


def _sc_scatter_s(isidx, inrm, osidx, onrm, zeros_s):
  mesh = plsc.VectorSubcoreMesh(core_axis_name="c", subcore_axis_name="s")
  f32 = jnp.float32
  BW = BLK * C
  SW = NPS * R
  SRPT = SW // NS

  @functools.partial(
      pl.kernel,
      out_type=jax.ShapeDtypeStruct((2 * SW,), f32),
      mesh=mesh,
      compiler_params=pltpu.CompilerParams(needs_layout_passes=False),
      scratch_types=[
          pltpu.VMEM((BW,), jnp.int32),
          pltpu.VMEM((BW,), f32),
          pltpu.VMEM((C,), jnp.int32),
          pltpu.VMEM((C,), jnp.int32),
          pltpu.VMEM((C,), f32),
          pltpu.VMEM((C,), f32),
          pltpu.VMEM_SHARED((SW,), f32),
          pltpu.SemaphoreType.DMA,
          pltpu.SemaphoreType.DMA,
      ],
  )
  def s_kernel(isidx_hbm, inrm_hbm, osidx_hbm, onrm_hbm,
               zs_hbm, out_hbm,
               sblk, nblk, fx0, fx1, vl0, vl1, stab, ss0, ss1):
    cid = lax.axis_index("c")
    sid = lax.axis_index("s")
    fxs = (fx0, fx1)
    vls = (vl0, vl1)
    ssems = (ss0, ss1)

    def body(sidx_hbm, nrm_hbm):
      pltpu.sync_copy(zs_hbm, stab.at[pl.ds(sid * SRPT, SRPT)])
      plsc.subcore_barrier()

      def prep(j, b):
        @pl.when(lax.rem(j, BLK) == 0)
        def _():
          blk_off = (sid * NCHUNK + j) * C
          pltpu.sync_copy(sidx_hbm.at[pl.ds(blk_off, BW)], sblk)
          pltpu.sync_copy(nrm_hbm.at[pl.ds(blk_off, BW)], nblk)
        off = lax.rem(j, BLK) * C
        for jj in range(C // 16):
          sl = pl.ds(off + jj * 16, 16)
          fxs[b][pl.ds(jj * 16, 16)] = sblk[sl]
          vls[b][pl.ds(jj * 16, 16)] = nblk[sl]
        pltpu.async_copy(vls[b], stab.at[fxs[b]], ssems[b], add=True)

      prep(jnp.int32(0), 0)

      def pair(k2, carry):
        for b in range(2):
          k = k2 * 2 + b
          nb = 1 - b

          @pl.when(k + 1 < NCHUNK)
          def _():
            @pl.when(k + 1 >= 2)
            def _():
              pltpu.make_async_copy(vls[nb], stab.at[fxs[nb]],
                                    ssems[nb]).wait()
            prep(k + 1, nb)
        return carry

      lax.fori_loop(0, NCHUNK // 2, pair, 0)
      pltpu.make_async_copy(vls[0], stab.at[fxs[0]], ssems[0]).wait()
      pltpu.make_async_copy(vls[1], stab.at[fxs[1]], ssems[1]).wait()
      plsc.subcore_barrier()
      pltpu.sync_copy(stab.at[pl.ds(sid * SRPT, SRPT)],
                      out_hbm.at[pl.ds(cid * SW + sid * SRPT, SRPT)])

    @pl.when(cid == 0)
    def _():
      body(isidx_hbm, inrm_hbm)

    @pl.when(cid == 1)
    def _():
      body(osidx_hbm, onrm_hbm)

  return s_kernel(isidx, inrm, osidx, onrm, zeros_s)


def _dot_t(x, w):
  return lax.dot_general(x, w, (((1,), (1,)), ((), ())),
                         preferred_element_type=jnp.float32)


def _dot(x, w):
  return lax.dot_general(x, w, (((1,), (0,)), ((), ())),
                         preferred_element_type=jnp.float32)


def _tc_combine_body(aic, air, aoc, aor, sin, sout, hc, hr,
                     win, wout, wloop, pic, pir, poc, por, lrc, lrr,
                     hnc_o, hnr_o):
  w_in = win[...]
  w_out = wout[...]
  w_loop = wloop[...]
  x = lrr[...]
  sp = jnp.maximum(x, 0.0) + jnp.log(1.0 + jnp.exp(-jnp.abs(x)))
  c3 = (_dot_t(aic[...], w_in) + _dot_t(aoc[...], w_out)
        + _dot(sin[...], pic[...]) - _dot(sout[...], poc[...])
        + _dot_t(hc[...] + lrc[...], w_loop))
  r3 = (_dot_t(air[...], jnp.abs(w_in)) + _dot_t(aor[...], jnp.abs(w_out))
        + _dot(sin[...], pir[...]) + _dot(sout[...], por[...])
        + _dot_t(hr[...] + sp, jnp.abs(w_loop)))
  c = c3 * (1.0 / 3.0)
  r = r3 * (1.0 / 3.0)
  lo = jnp.maximum(c - r, 0.0)
  hi = jnp.maximum(c + r, 0.0)
  hnc_o[...] = (hi + lo) * 0.5
  hnr_o[...] = (hi - lo) * 0.5


RP = 256


def _tc_combine(a_in_c, a_in_r, a_out_c, a_out_r, S_in, S_out, H_c, H_r,
                W_in, W_out, W_loop, P_in_c, P_in_r, P_out_c, P_out_r,
                loop_rel_c, loop_rel_r):
  blk = 2000
  grid = (N // blk,)
  row_spec = pl.BlockSpec((blk, D), lambda i: (i, 0))
  s_spec = pl.BlockSpec((blk, RP), lambda i: (i, 0))
  w_spec = pl.BlockSpec((D, D), lambda i: (0, 0))
  p_spec = pl.BlockSpec((RP, D), lambda i: (0, 0))
  v_spec = pl.BlockSpec((1, D), lambda i: (0, 0))
  return pl.pallas_call(
      _tc_combine_body,
      grid=grid,
      in_specs=([row_spec] * 4 + [s_spec] * 2 + [row_spec] * 2
                + [w_spec] * 3 + [p_spec] * 4 + [v_spec] * 2),
      out_specs=[row_spec, row_spec],
      out_shape=[jax.ShapeDtypeStruct((N, D), jnp.float32)] * 2,
  )(a_in_c, a_in_r, a_out_c, a_out_r, S_in, S_out, H_c, H_r,
    W_in, W_out, W_loop, P_in_c, P_in_r, P_out_c, P_out_r,
    loop_rel_c, loop_rel_r)


def _tc_rel_body(rc, rr, wr, win, wout, orc_o, orr_o, pic_o, pir_o,
                 poc_o, por_o):
  w = wr[...]
  rcv = rc[...]
  rrv = rr[...]
  orc_o[...] = _dot_t(rcv, w)
  orr_o[...] = _dot_t(rrv, jnp.abs(w))
  pic_o[...] = _dot_t(rcv, win[...])
  pir_o[...] = _dot_t(rrv, jnp.abs(win[...]))
  poc_o[...] = _dot_t(rcv, wout[...])
  por_o[...] = _dot_t(rrv, jnp.abs(wout[...]))


def _tc_rel(rel_c, rel_r, W_rel, W_in, W_out):
  return pl.pallas_call(
      _tc_rel_body,
      out_shape=[jax.ShapeDtypeStruct((R, D), jnp.float32)] * 6,
  )(rel_c, rel_r, W_rel, W_in, W_out)


def kernel(H_c, H_r, rel_c, rel_r, in_row, in_col, in_type, in_norm,
           out_row, out_col, out_type, out_norm, loop_row, loop_col,
           W_in, W_out, W_loop, W_rel, loop_rel_c, loop_rel_r):
  zeros_tile = jnp.zeros((RPT, D), jnp.float32)
  in_row = in_row.astype(jnp.int32)
  in_col = in_col.astype(jnp.int32)
  in_type = in_type.astype(jnp.int32)
  out_row = out_row.astype(jnp.int32)
  out_col = out_col.astype(jnp.int32)
  out_type = out_type.astype(jnp.int32)
  H_all = jnp.concatenate([H_c, H_r], axis=0)
  pirow = _pad_edges(in_row)
  pinrm = _pad_edges(in_norm)
  porow = _pad_edges(out_row)
  ponrm = _pad_edges(out_norm)
  outs = _sc_aggregate(
      H_all, _pad_edges(in_col), pirow, pinrm,
      _pad_edges(out_col), porow, ponrm, zeros_tile)
  zeros_s = jnp.zeros(((NPS // NS) * R,), jnp.float32)
  s_flat = _sc_scatter_s(_pad_edges(in_row * R + in_type), pinrm,
                         _pad_edges(out_row * R + out_type), ponrm, zeros_s)
  s_pad = jnp.pad(s_flat.reshape(2, NPS, R)[:, :N],
                  ((0, 0), (0, 0), (0, RP - R)))
  a_in_c = outs[:N]
  a_in_r = outs[NP:NP + N]
  a_out_c = outs[2 * NP:2 * NP + N]
  a_out_r = outs[3 * NP:3 * NP + N]
  new_rel_c, new_rel_r, p_in_c, p_in_r, p_out_c, p_out_r = _tc_rel(
      rel_c, rel_r, W_rel, W_in, W_out)
  pad_p = lambda p: jnp.pad(p, ((0, RP - R), (0, 0)))
  Hn_c, Hn_r = _tc_combine(
      a_in_c, a_in_r, a_out_c, a_out_r, s_pad[0], s_pad[1], H_c, H_r,
      W_in, W_out, W_loop, pad_p(p_in_c), pad_p(p_in_r),
      pad_p(p_out_c), pad_p(p_out_r), loop_rel_c, loop_rel_r)
  return Hn_c, Hn_r, new_rel_c, new_rel_r


# speedup vs baseline: 5.5764x; 1.0040x over previous
import functools

import jax
import jax.numpy as jnp
from jax import lax
from jax.experimental import pallas as pl
from jax.experimental.pallas import tpu as pltpu
from jax.experimental.pallas import tpu_sc as plsc

N = 10000
E = 320000
D = 128
R = 200

NC = 2
NS = 16
EPT = E // NS
C = 48
EPTP = 20160
NCHUNK = EPTP // C
BLK = 20
NP = 10112
RPT = NP // NS


def _pad_edges(x):
  return jnp.pad(x.reshape(NS, EPT), ((0, 0), (0, EPTP - EPT))).reshape(-1)


def _sc_aggregate(H_all, icol, irow, inrm, ocol, orow, onrm, zeros_tile):
  mesh = plsc.VectorSubcoreMesh(core_axis_name="c", subcore_axis_name="s")
  f32 = jnp.float32
  BW = BLK * C

  @functools.partial(
      pl.kernel,
      out_type=jax.ShapeDtypeStruct((4 * NP, D), f32),
      mesh=mesh,
      compiler_params=pltpu.CompilerParams(needs_layout_passes=False),
      scratch_types=[
          pltpu.VMEM((BW,), jnp.int32),
          pltpu.VMEM((BW,), jnp.int32),
          pltpu.VMEM((BW,), f32),
          pltpu.VMEM((C,), jnp.int32),
          pltpu.VMEM((C,), jnp.int32),
          pltpu.VMEM((C,), jnp.int32),
          pltpu.VMEM((C,), jnp.int32),
          pltpu.VMEM((C + 16,), f32),
          pltpu.VMEM((C + 16,), f32),
          pltpu.VMEM((C, D), f32),
          pltpu.VMEM((C, D), f32),
          pltpu.VMEM((C, D), f32),
          pltpu.VMEM_SHARED((NP, D), f32),
          pltpu.SemaphoreType.DMA,
          pltpu.SemaphoreType.DMA,
          pltpu.SemaphoreType.DMA,
      ],
  )
  def sc_kernel(h_hbm, icol_hbm, irow_hbm, inrm_hbm,
                ocol_hbm, orow_hbm, onrm_hbm, z_hbm, out_hbm,
                cblk, rblk, nblk, row0, row1, col0, col1,
                nrm0, nrm1, h0, h1, msg, acc, sh0, sh1, ss):
    cid = lax.axis_index("c")
    sid = lax.axis_index("s")
    iota16 = lax.broadcasted_iota(jnp.int32, (16,), 0)
    rows = (row0, row1)
    cols = (col0, col1)
    nrms = (nrm0, nrm1)
    hbufs = (h0, h1)
    hsems = (sh0, sh1)

    def body(p, col_hbm, row_hbm, nrm_hbm):
      seg = cid * 2 + p
      coloff = p * N
      out_off = seg * NP
      pltpu.sync_copy(z_hbm, acc.at[pl.ds(sid * RPT, RPT)])
      plsc.subcore_barrier()

      def prep(j, b):
        @pl.when(lax.rem(j, BLK) == 0)
        def _():
          blk_off = (sid * NCHUNK + j) * C
          pltpu.sync_copy(col_hbm.at[pl.ds(blk_off, BW)], cblk)
          pltpu.sync_copy(row_hbm.at[pl.ds(blk_off, BW)], rblk)
          pltpu.sync_copy(nrm_hbm.at[pl.ds(blk_off, BW)], nblk)
        off = lax.rem(j, BLK) * C
        for jj in range(C // 16):
          sl = pl.ds(off + jj * 16, 16)
          cols[b][pl.ds(jj * 16, 16)] = cblk[sl] + coloff
          rows[b][pl.ds(jj * 16, 16)] = rblk[sl]
          nrms[b][pl.ds(jj * 16, 16)] = nblk[sl]
        pltpu.async_copy(h_hbm.at[cols[b]], hbufs[b], hsems[b])

      def compute(k, b):
        pltpu.make_async_copy(h_hbm.at[cols[b]], hbufs[b], hsems[b]).wait()

        @plsc.parallel_loop(0, C, step=1, unroll=8)
        def _(e):
          n16 = jnp.full((16,), nrms[b][pl.ds(e, 16)][0], f32)
          for dc in range(D // 16):
            h16 = hbufs[b][e, pl.ds(dc * 16, 16)]
            msg[e, pl.ds(dc * 16, 16)] = h16 * n16

        pltpu.async_copy(msg, acc.at[rows[b]], ss, add=True)

      prep(jnp.int32(0), 0)

      def pair(k2, carry):
        for b in range(2):
          k = k2 * 2 + b
          nb = 1 - b

          @pl.when(k >= 1)
          def _():
            pltpu.make_async_copy(msg, acc.at[rows[b]], ss).wait()

          @pl.when(k + 1 < NCHUNK)
          def _():
            prep(k + 1, nb)
          compute(k, b)
        return carry

      lax.fori_loop(0, NCHUNK // 2, pair, 0)
      pltpu.make_async_copy(msg, acc.at[rows[1]], ss).wait()
      plsc.subcore_barrier()
      pltpu.sync_copy(acc.at[pl.ds(sid * RPT, RPT)],
                      out_hbm.at[pl.ds(out_off + sid * RPT, RPT)])
      plsc.subcore_barrier()

    def do_pass(p, carry):
      @pl.when(cid == 0)
      def _():
        body(p, icol_hbm, irow_hbm, inrm_hbm)

      @pl.when(cid == 1)
      def _():
        body(p, ocol_hbm, orow_hbm, onrm_hbm)
      return carry

    lax.fori_loop(0, 2, do_pass, 0)

  return sc_kernel(H_all, icol, irow, inrm,
                   ocol, orow, onrm, zeros_tile)


NPS = 10240


def _sc_scatter_s(isidx, inrm, osidx, onrm, zeros_s):
  mesh = plsc.VectorSubcoreMesh(core_axis_name="c", subcore_axis_name="s")
  f32 = jnp.float32
  BW = BLK * C
  SW = NPS * R
  SRPT = SW // NS

  @functools.partial(
      pl.kernel,
      out_type=jax.ShapeDtypeStruct((2 * SW,), f32),
      mesh=mesh,
      compiler_params=pltpu.CompilerParams(needs_layout_passes=False),
      scratch_types=[
          pltpu.VMEM((BW,), jnp.int32),
          pltpu.VMEM((BW,), f32),
          pltpu.VMEM((C,), jnp.int32),
          pltpu.VMEM((C,), jnp.int32),
          pltpu.VMEM((C,), f32),
          pltpu.VMEM((C,), f32),
          pltpu.VMEM_SHARED((SW,), f32),
          pltpu.SemaphoreType.DMA,
          pltpu.SemaphoreType.DMA,
      ],
  )
  def s_kernel(isidx_hbm, inrm_hbm, osidx_hbm, onrm_hbm,
               zs_hbm, out_hbm,
               sblk, nblk, fx0, fx1, vl0, vl1, stab, ss0, ss1):
    cid = lax.axis_index("c")
    sid = lax.axis_index("s")
    fxs = (fx0, fx1)
    vls = (vl0, vl1)
    ssems = (ss0, ss1)

    def body(sidx_hbm, nrm_hbm):
      pltpu.sync_copy(zs_hbm, stab.at[pl.ds(sid * SRPT, SRPT)])
      plsc.subcore_barrier()

      def prep(j, b):
        @pl.when(lax.rem(j, BLK) == 0)
        def _():
          blk_off = (sid * NCHUNK + j) * C
          pltpu.sync_copy(sidx_hbm.at[pl.ds(blk_off, BW)], sblk)
          pltpu.sync_copy(nrm_hbm.at[pl.ds(blk_off, BW)], nblk)
        off = lax.rem(j, BLK) * C
        for jj in range(C // 16):
          sl = pl.ds(off + jj * 16, 16)
          fxs[b][pl.ds(jj * 16, 16)] = sblk[sl]
          vls[b][pl.ds(jj * 16, 16)] = nblk[sl]
        pltpu.async_copy(vls[b], stab.at[fxs[b]], ssems[b], add=True)

      prep(jnp.int32(0), 0)

      def pair(k2, carry):
        for b in range(2):
          k = k2 * 2 + b
          nb = 1 - b

          @pl.when(k + 1 < NCHUNK)
          def _():
            @pl.when(k + 1 >= 2)
            def _():
              pltpu.make_async_copy(vls[nb], stab.at[fxs[nb]],
                                    ssems[nb]).wait()
            prep(k + 1, nb)
        return carry

      lax.fori_loop(0, NCHUNK // 2, pair, 0)
      pltpu.make_async_copy(vls[0], stab.at[fxs[0]], ssems[0]).wait()
      pltpu.make_async_copy(vls[1], stab.at[fxs[1]], ssems[1]).wait()
      plsc.subcore_barrier()
      pltpu.sync_copy(stab.at[pl.ds(sid * SRPT, SRPT)],
                      out_hbm.at[pl.ds(cid * SW + sid * SRPT, SRPT)])

    @pl.when(cid == 0)
    def _():
      body(isidx_hbm, inrm_hbm)

    @pl.when(cid == 1)
    def _():
      body(osidx_hbm, onrm_hbm)

  return s_kernel(isidx, inrm, osidx, onrm, zeros_s)


def _dot_t(x, w):
  return lax.dot_general(x, w, (((1,), (1,)), ((), ())),
                         preferred_element_type=jnp.float32)


def _dot(x, w):
  return lax.dot_general(x, w, (((1,), (0,)), ((), ())),
                         preferred_element_type=jnp.float32)


def _tc_combine_body(aic, air, aoc, aor, sin, sout, hc, hr,
                     win, wout, wloop, pic, pir, poc, por, lrc, lrr,
                     hnc_o, hnr_o):
  w_in = win[...]
  w_out = wout[...]
  w_loop = wloop[...]
  x = lrr[...]
  sp = jnp.maximum(x, 0.0) + jnp.log(1.0 + jnp.exp(-jnp.abs(x)))
  c3 = (_dot_t(aic[...], w_in) + _dot_t(aoc[...], w_out)
        + _dot(sin[...], pic[...]) - _dot(sout[...], poc[...])
        + _dot_t(hc[...] + lrc[...], w_loop))
  r3 = (_dot_t(air[...], jnp.abs(w_in)) + _dot_t(aor[...], jnp.abs(w_out))
        + _dot(sin[...], pir[...]) + _dot(sout[...], por[...])
        + _dot_t(hr[...] + sp, jnp.abs(w_loop)))
  c = c3 * (1.0 / 3.0)
  r = r3 * (1.0 / 3.0)
  lo = jnp.maximum(c - r, 0.0)
  hi = jnp.maximum(c + r, 0.0)
  hnc_o[...] = (hi + lo) * 0.5
  hnr_o[...] = (hi - lo) * 0.5


RP = 256


def _tc_combine(a_in_c, a_in_r, a_out_c, a_out_r, S_in, S_out, H_c, H_r,
                W_in, W_out, W_loop, P_in_c, P_in_r, P_out_c, P_out_r,
                loop_rel_c, loop_rel_r):
  blk = 2000
  grid = (N // blk,)
  row_spec = pl.BlockSpec((blk, D), lambda i: (i, 0))
  s_spec = pl.BlockSpec((blk, RP), lambda i: (i, 0))
  w_spec = pl.BlockSpec((D, D), lambda i: (0, 0))
  p_spec = pl.BlockSpec((RP, D), lambda i: (0, 0))
  v_spec = pl.BlockSpec((1, D), lambda i: (0, 0))
  return pl.pallas_call(
      _tc_combine_body,
      grid=grid,
      in_specs=([row_spec] * 4 + [s_spec] * 2 + [row_spec] * 2
                + [w_spec] * 3 + [p_spec] * 4 + [v_spec] * 2),
      out_specs=[row_spec, row_spec],
      out_shape=[jax.ShapeDtypeStruct((N, D), jnp.float32)] * 2,
  )(a_in_c, a_in_r, a_out_c, a_out_r, S_in, S_out, H_c, H_r,
    W_in, W_out, W_loop, P_in_c, P_in_r, P_out_c, P_out_r,
    loop_rel_c, loop_rel_r)


def _tc_rel_body(rc, rr, wr, win, wout, orc_o, orr_o, pic_o, pir_o,
                 poc_o, por_o):
  w = wr[...]
  rcv = rc[...]
  rrv = rr[...]
  orc_o[...] = _dot_t(rcv, w)
  orr_o[...] = _dot_t(rrv, jnp.abs(w))
  pic_o[...] = _dot_t(rcv, win[...])
  pir_o[...] = _dot_t(rrv, jnp.abs(win[...]))
  poc_o[...] = _dot_t(rcv, wout[...])
  por_o[...] = _dot_t(rrv, jnp.abs(wout[...]))


def _tc_rel(rel_c, rel_r, W_rel, W_in, W_out):
  return pl.pallas_call(
      _tc_rel_body,
      out_shape=[jax.ShapeDtypeStruct((R, D), jnp.float32)] * 6,
  )(rel_c, rel_r, W_rel, W_in, W_out)


def kernel(H_c, H_r, rel_c, rel_r, in_row, in_col, in_type, in_norm,
           out_row, out_col, out_type, out_norm, loop_row, loop_col,
           W_in, W_out, W_loop, W_rel, loop_rel_c, loop_rel_r):
  zeros_tile = jnp.zeros((RPT, D), jnp.float32)
  in_row = in_row.astype(jnp.int32)
  in_col = in_col.astype(jnp.int32)
  in_type = in_type.astype(jnp.int32)
  out_row = out_row.astype(jnp.int32)
  out_col = out_col.astype(jnp.int32)
  out_type = out_type.astype(jnp.int32)
  H_all = jnp.concatenate([H_c, H_r], axis=0)
  pirow = _pad_edges(in_row)
  pinrm = _pad_edges(in_norm)
  porow = _pad_edges(out_row)
  ponrm = _pad_edges(out_norm)
  outs = _sc_aggregate(
      H_all, _pad_edges(in_col), pirow, pinrm,
      _pad_edges(out_col), porow, ponrm, zeros_tile)
  zeros_s = jnp.zeros(((NPS // NS) * R,), jnp.float32)
  s_flat = _sc_scatter_s(_pad_edges(in_row * R + in_type), pinrm,
                         _pad_edges(out_row * R + out_type), ponrm, zeros_s)
  s_pad = jnp.pad(s_flat.reshape(2, NPS, R)[:, :N],
                  ((0, 0), (0, 0), (0, RP - R)))
  a_in_c = outs[:N]
  a_in_r = outs[NP:NP + N]
  a_out_c = outs[2 * NP:2 * NP + N]
  a_out_r = outs[3 * NP:3 * NP + N]
  new_rel_c, new_rel_r, p_in_c, p_in_r, p_out_c, p_out_r = _tc_rel(
      rel_c, rel_r, W_rel, W_in, W_out)
  pad_p = lambda p: jnp.pad(p, ((0, RP - R), (0, 0)))
  Hn_c, Hn_r = _tc_combine(
      a_in_c, a_in_r, a_out_c, a_out_r, s_pad[0], s_pad[1], H_c, H_r,
      W_in, W_out, W_loop, pad_p(p_in_c), pad_p(p_in_r),
      pad_p(p_out_c), pad_p(p_out_r), loop_rel_c, loop_rel_r)
  return Hn_c, Hn_r, new_rel_c, new_rel_r
